# Initial kernel scaffold; baseline (speedup 1.0000x reference)
import jax
import jax.numpy as jnp
from jax.experimental import pallas as pl


def kernel(logits, noise_std, training):
    raise NotImplementedError("write your pallas kernel here")


# speedup vs baseline: 1.4780x; 1.4780x over previous
import functools

import jax
import jax.numpy as jnp
from jax import lax
from jax.experimental import pallas as pl
from jax.experimental.pallas import tpu as pltpu
from jax.experimental.pallas import tpu_sc as plsc

NUM_EXPERTS = 64
TOP_K = 8
B = 32768

_NC = 2
_NS = 16
_NW = _NC * _NS
_TOK_W = B // _NW
_LOG_W = _TOK_W * NUM_EXPERTS
_OUT_W = _TOK_W * TOP_K


@functools.partial(
    pl.kernel,
    out_type=(
        jax.ShapeDtypeStruct((B * TOP_K,), jnp.float32),
        jax.ShapeDtypeStruct((B * TOP_K,), jnp.int32),
    ),
    mesh=plsc.VectorSubcoreMesh(core_axis_name="c", subcore_axis_name="s"),
    compiler_params=pltpu.CompilerParams(needs_layout_passes=False),
    scratch_types=[
        pltpu.VMEM((_LOG_W,), jnp.float32),
        pltpu.VMEM((_OUT_W + 8,), jnp.float32),
        pltpu.VMEM((_OUT_W + 8,), jnp.int32),
    ],
)
def _router(logits_hbm, vals_hbm, idx_hbm, lbuf, vbuf, ibuf):
    wid = lax.axis_index("s") * _NC + lax.axis_index("c")
    pltpu.sync_copy(logits_hbm.at[pl.ds(wid * _LOG_W, _LOG_W)], lbuf)

    lane = lax.iota(jnp.int32, 16)
    mask8 = lane < 8

    @plsc.parallel_loop(0, _TOK_W, unroll=4)
    def _token(t):
        off = t * NUM_EXPERTS
        e0 = jnp.exp(lbuf[pl.ds(off, 16)])
        e1 = jnp.exp(lbuf[pl.ds(off + 16, 16)])
        e2 = jnp.exp(lbuf[pl.ds(off + 32, 16)])
        e3 = jnp.exp(lbuf[pl.ds(off + 48, 16)])
        zc = plsc.cumsum((e0 + e1) + (e2 + e3))
        z = zc[jnp.full((16,), 15, jnp.int32)]

        k0, v0 = plsc.sort_key_val(e0, lane, descending=True)
        k1, v1 = plsc.sort_key_val(e1, lane + 16, descending=False)
        k2, v2 = plsc.sort_key_val(e2, lane + 32, descending=True)
        k3, v3 = plsc.sort_key_val(e3, lane + 48, descending=False)

        m01k, m01v = plsc.sort_key_val(
            jnp.where(mask8, k0, k1), jnp.where(mask8, v0, v1), descending=True)
        m23k, m23v = plsc.sort_key_val(
            jnp.where(mask8, k2, k3), jnp.where(mask8, v2, v3), descending=False)

        fk, fv = plsc.sort_key_val(
            jnp.where(mask8, m01k, m23k), jnp.where(mask8, m01v, m23v),
            descending=True)

        s8 = plsc.cumsum(fk)[jnp.full((16,), TOP_K - 1, jnp.int32)]
        r = 1.0 / (s8 + 1e-6 * z)
        obase = t * TOP_K
        plsc.store_compressed(vbuf.at[pl.ds(obase, 16)], fk * r, mask=mask8)
        plsc.store_compressed(ibuf.at[pl.ds(obase, 16)], fv, mask=mask8)

    pltpu.sync_copy(vbuf.at[pl.ds(0, _OUT_W)],
                    vals_hbm.at[pl.ds(wid * _OUT_W, _OUT_W)])
    pltpu.sync_copy(ibuf.at[pl.ds(0, _OUT_W)],
                    idx_hbm.at[pl.ds(wid * _OUT_W, _OUT_W)])


def kernel(logits, noise_std, training):
    del noise_std, training
    vals, idx = _router(logits.reshape(-1))
    return vals.reshape(B, TOP_K), idx.reshape(B, TOP_K)
